# baseline (speedup 1.0000x reference)
import functools

import jax
import jax.numpy as jnp
from jax import lax
from jax.experimental import pallas as pl
from jax.experimental.pallas import tpu as pltpu
from jax.experimental.pallas import tpu_sc as plsc

N = 10000
P = 10112
E = 320000
NC, NS = 2, 16
B = 128
EP = 327680
NB_AGG = EP // (NS * B)
NB_DEG = EP // (NC * NS * B)
NCHUNK = 4
NB_IN = NB_AGG // NCHUNK
NB2 = NB_IN // 2
RPT = P // NS


@functools.cache
def _sc_kernels():
    mesh = plsc.VectorSubcoreMesh(core_axis_name="c", subcore_axis_name="s",
                                  num_cores=NC, num_subcores=NS)

    @functools.partial(
        pl.kernel,
        out_type=jax.ShapeDtypeStruct((NC, P, 128), jnp.float32),
        mesh=mesh,
        scratch_types=[
            pltpu.VMEM((NB_DEG, B), jnp.int32),
            pltpu.VMEM((B, 128), jnp.float32),
            pltpu.VMEM_SHARED((P, 128), jnp.float32),
        ],
    )
    def deg_kernel(dst_hbm, ones_hbm, zeros_hbm, out_hbm, dst_v, ones_v, acc_sh):
        c = lax.axis_index("c")
        s = lax.axis_index("s")
        pltpu.sync_copy(dst_hbm.at[c, s], dst_v)
        pltpu.sync_copy(ones_hbm, ones_v)
        pltpu.sync_copy(zeros_hbm, acc_sh.at[pl.ds(s * RPT, RPT)])
        plsc.subcore_barrier()

        def body(j, _):
            pltpu.sync_copy(ones_v, acc_sh.at[dst_v.at[j]], add=True)
            return 0
        lax.fori_loop(0, NB_DEG, body, 0)

        plsc.subcore_barrier()
        pltpu.sync_copy(acc_sh.at[pl.ds(s * RPT, RPT)],
                        out_hbm.at[c, pl.ds(s * RPT, RPT)])

    @functools.partial(
        pl.kernel,
        out_type=jax.ShapeDtypeStruct((NC, P, 128), jnp.float32),
        mesh=mesh,
        scratch_types=[
            pltpu.VMEM((NB_IN + 2, B), jnp.int32),
            pltpu.VMEM((NB_IN, B), jnp.int32),
            pltpu.VMEM((B, 128), jnp.float32),
            pltpu.VMEM((B, 128), jnp.float32),
            pltpu.VMEM_SHARED((P, 128), jnp.float32),
            pltpu.SemaphoreType.DMA,
            pltpu.SemaphoreType.DMA,
        ],
    )
    def agg_kernel(table_hbm, src_hbm, dst_hbm, out_hbm,
                   src_v, dst_v, buf0, buf1, acc_sh, semg0, semg1):
        c = lax.axis_index("c")
        s = lax.axis_index("s")
        pltpu.sync_copy(table_hbm.at[pl.ds(c * P + s * RPT, RPT)],
                        acc_sh.at[pl.ds(s * RPT, RPT)])
        plsc.subcore_barrier()

        def chunk(ch, _):
            pltpu.sync_copy(src_hbm.at[c, s, ch], src_v)
            pltpu.sync_copy(dst_hbm.at[s, ch], dst_v)
            pltpu.async_copy(table_hbm.at[src_v.at[0]], buf0, semg0)
            pltpu.async_copy(table_hbm.at[src_v.at[1]], buf1, semg1)

            def body(k, _):
                j0 = 2 * k
                j1 = j0 + 1
                pltpu.make_async_copy(table_hbm.at[src_v.at[j0]],
                                      buf0, semg0).wait()
                pltpu.sync_copy(buf0, acc_sh.at[dst_v.at[j0]], add=True)
                pltpu.async_copy(table_hbm.at[src_v.at[j0 + 2]], buf0, semg0)
                pltpu.make_async_copy(table_hbm.at[src_v.at[j1]],
                                      buf1, semg1).wait()
                pltpu.sync_copy(buf1, acc_sh.at[dst_v.at[j1]], add=True)
                pltpu.async_copy(table_hbm.at[src_v.at[j1 + 2]], buf1, semg1)
                return 0
            lax.fori_loop(0, NB2, body, 0)
            pltpu.make_async_copy(table_hbm.at[src_v.at[0]],
                                  buf0, semg0).wait()
            pltpu.make_async_copy(table_hbm.at[src_v.at[1]],
                                  buf1, semg1).wait()
            return 0
        lax.fori_loop(0, NCHUNK, chunk, 0)

        plsc.subcore_barrier()
        pltpu.sync_copy(acc_sh.at[pl.ds(s * RPT, RPT)],
                        out_hbm.at[c, pl.ds(s * RPT, RPT)])

    return deg_kernel, agg_kernel


R = 2528
R_OUT = 2000


def _dinv_of(degp_ref):
    return lax.rsqrt(degp_ref[0, :, 0] + degp_ref[1, :, 0] + 1.0)


def _mm_scale_body(x_ref, w_ref, degp_ref, out_ref):
    dinv = _dinv_of(degp_ref)
    h = jnp.dot(x_ref[...], w_ref[...], preferred_element_type=jnp.float32)
    hp = h * dinv[:, None]
    out_ref[0] = hp[:, :128]
    out_ref[1] = hp[:, 128:]


def _mid_body(agg_ref, degp_ref, b_ref, w_ref, out_ref):
    dinv = _dinv_of(degp_ref)
    h = jnp.concatenate([agg_ref[0], agg_ref[1]], axis=1) * dinv[:, None]
    h = jnp.maximum(h + b_ref[...], 0.0)
    h2 = jnp.dot(h, w_ref[...], preferred_element_type=jnp.float32)
    hp = h2 * dinv[:, None]
    out_ref[0] = hp[:, :128]
    out_ref[1] = hp[:, 128:]


def _out_body(agg_ref, degp_ref, b_ref, out_ref):
    dinv = _dinv_of(degp_ref)
    h = jnp.concatenate([agg_ref[0], agg_ref[1]], axis=1) * dinv[:, None]
    out_ref[...] = jnp.maximum(h + b_ref[...], 0.0)


_mm_call = pl.pallas_call(
    _mm_scale_body,
    grid=(P // R,),
    in_specs=[
        pl.BlockSpec((R, 128), lambda i: (i, 0)),
        pl.BlockSpec((128, 256), lambda i: (0, 0)),
        pl.BlockSpec((2, R, 128), lambda i: (0, i, 0)),
    ],
    out_specs=pl.BlockSpec((2, R, 128), lambda i: (0, i, 0)),
    out_shape=jax.ShapeDtypeStruct((2, P, 128), jnp.float32),
)

_mid_call = pl.pallas_call(
    _mid_body,
    grid=(P // R,),
    in_specs=[
        pl.BlockSpec((2, R, 128), lambda i: (0, i, 0)),
        pl.BlockSpec((2, R, 128), lambda i: (0, i, 0)),
        pl.BlockSpec((1, 256), lambda i: (0, 0)),
        pl.BlockSpec((256, 256), lambda i: (0, 0)),
    ],
    out_specs=pl.BlockSpec((2, R, 128), lambda i: (0, i, 0)),
    out_shape=jax.ShapeDtypeStruct((2, P, 128), jnp.float32),
)

_out_call = pl.pallas_call(
    _out_body,
    grid=(N // R_OUT,),
    in_specs=[
        pl.BlockSpec((2, R_OUT, 128), lambda i: (0, i, 0)),
        pl.BlockSpec((2, R_OUT, 128), lambda i: (0, i, 0)),
        pl.BlockSpec((1, 256), lambda i: (0, 0)),
    ],
    out_specs=pl.BlockSpec((R_OUT, 256), lambda i: (i, 0)),
    out_shape=jax.ShapeDtypeStruct((N, 256), jnp.float32),
)


def kernel(x, edge_index, W1, b1, W2, b2):
    src = edge_index[0].astype(jnp.int32)
    dst = edge_index[1].astype(jnp.int32)
    pad = EP - E
    src_p = jnp.concatenate([src, jnp.zeros((pad,), jnp.int32)])
    dst_p = jnp.concatenate([dst, jnp.full((pad,), N, jnp.int32)])
    dst_deg = dst_p.reshape(NC, NS, NB_DEG, B)
    dst_agg = dst_p.reshape(NS, NCHUNK, NB_IN, B)
    src_agg = jnp.stack([src_p, src_p + P]).reshape(NC, NS, NCHUNK, NB_IN, B)
    src_agg = jnp.pad(src_agg, ((0, 0), (0, 0), (0, 0), (0, 2), (0, 0)))
    xp = jnp.pad(x, ((0, P - N), (0, 0)))

    deg_kernel, agg_kernel = _sc_kernels()
    ones_hbm = jnp.ones((B, 128), jnp.float32)
    zeros_hbm = jnp.zeros((RPT, 128), jnp.float32)
    degp = deg_kernel(dst_deg, ones_hbm, zeros_hbm)
    t1 = _mm_call(xp, W1, degp)
    agg1 = agg_kernel(t1.reshape(NC * P, 128), src_agg, dst_agg)
    t2 = _mid_call(agg1, degp, b1.reshape(1, 256), W2)
    agg2 = agg_kernel(t2.reshape(NC * P, 128), src_agg, dst_agg)
    return _out_call(agg2, degp, b2.reshape(1, 256))


# speedup vs baseline: 2.3298x; 2.3298x over previous
import functools

import jax
import jax.numpy as jnp
from jax import lax
from jax.experimental import pallas as pl
from jax.experimental.pallas import tpu as pltpu
from jax.experimental.pallas import tpu_sc as plsc

N = 10000
P = 10112
E = 320000
NC, NS = 2, 16
B = 128
EP = 327680
NB_AGG = EP // (NS * B)
NB_DEG = EP // (NC * NS * B)
NCHUNK = 2
NB_IN = NB_AGG // NCHUNK
RPT = P // NS


@functools.cache
def _sc_kernels():
    mesh = plsc.VectorSubcoreMesh(core_axis_name="c", subcore_axis_name="s",
                                  num_cores=NC, num_subcores=NS)

    @functools.partial(
        pl.kernel,
        out_type=jax.ShapeDtypeStruct((NC, P, 128), jnp.float32),
        mesh=mesh,
        scratch_types=[
            pltpu.VMEM((NB_DEG, B), jnp.int32),
            pltpu.VMEM((B, 128), jnp.float32),
            pltpu.VMEM_SHARED((P, 128), jnp.float32),
        ],
    )
    def deg_kernel(dst_hbm, ones_hbm, zeros_hbm, out_hbm, dst_v, ones_v, acc_sh):
        c = lax.axis_index("c")
        s = lax.axis_index("s")
        pltpu.sync_copy(dst_hbm.at[c, s], dst_v)
        pltpu.sync_copy(ones_hbm, ones_v)
        pltpu.sync_copy(zeros_hbm, acc_sh.at[pl.ds(s * RPT, RPT)])
        plsc.subcore_barrier()

        def body(j, _):
            pltpu.sync_copy(ones_v, acc_sh.at[dst_v.at[j]], add=True)
            return 0
        lax.fori_loop(0, NB_DEG, body, 0)

        plsc.subcore_barrier()
        pltpu.sync_copy(acc_sh.at[pl.ds(s * RPT, RPT)],
                        out_hbm.at[c, pl.ds(s * RPT, RPT)])

    @functools.partial(
        pl.kernel,
        out_type=jax.ShapeDtypeStruct((NC, P, 128), jnp.float32),
        mesh=mesh,
        scratch_types=[
            pltpu.VMEM((NB_IN, B), jnp.int32),
            pltpu.VMEM((NB_IN, B), jnp.int32),
            pltpu.VMEM((B, 128), jnp.float32),
            pltpu.VMEM_SHARED((P, 128), jnp.float32),
            pltpu.SemaphoreType.DMA,
        ],
    )
    def agg_kernel(table_hbm, src_hbm, dst_hbm, out_hbm,
                   src_v, dst_v, buf0, acc_sh, semg0):
        c = lax.axis_index("c")
        s = lax.axis_index("s")
        pltpu.sync_copy(table_hbm.at[pl.ds(c * P + s * RPT, RPT)],
                        acc_sh.at[pl.ds(s * RPT, RPT)])
        plsc.subcore_barrier()

        def chunk(ch, _):
            pltpu.sync_copy(src_hbm.at[c, s, ch], src_v)
            pltpu.sync_copy(dst_hbm.at[s, ch], dst_v)

            def body(j, _):
                pltpu.async_copy(table_hbm.at[src_v.at[j]], buf0, semg0).wait()
                pltpu.sync_copy(buf0, acc_sh.at[dst_v.at[j]], add=True)
                return 0
            lax.fori_loop(0, NB_IN, body, 0)
            return 0
        lax.fori_loop(0, NCHUNK, chunk, 0)

        plsc.subcore_barrier()
        pltpu.sync_copy(acc_sh.at[pl.ds(s * RPT, RPT)],
                        out_hbm.at[c, pl.ds(s * RPT, RPT)])

    return deg_kernel, agg_kernel


R = 2528
R_OUT = 2000


def _dinv_of(degp_ref):
    return lax.rsqrt(degp_ref[0, :, 0] + degp_ref[1, :, 0] + 1.0)


def _mm_scale_body(x_ref, w_ref, degp_ref, out_ref):
    dinv = _dinv_of(degp_ref)
    h = jnp.dot(x_ref[...], w_ref[...], preferred_element_type=jnp.float32)
    hp = h * dinv[:, None]
    out_ref[0] = hp[:, :128]
    out_ref[1] = hp[:, 128:]


def _mid_body(agg_ref, degp_ref, b_ref, w_ref, out_ref):
    dinv = _dinv_of(degp_ref)
    h = jnp.concatenate([agg_ref[0], agg_ref[1]], axis=1) * dinv[:, None]
    h = jnp.maximum(h + b_ref[...], 0.0)
    h2 = jnp.dot(h, w_ref[...], preferred_element_type=jnp.float32)
    hp = h2 * dinv[:, None]
    out_ref[0] = hp[:, :128]
    out_ref[1] = hp[:, 128:]


def _out_body(agg_ref, degp_ref, b_ref, out_ref):
    dinv = _dinv_of(degp_ref)
    h = jnp.concatenate([agg_ref[0], agg_ref[1]], axis=1) * dinv[:, None]
    out_ref[...] = jnp.maximum(h + b_ref[...], 0.0)


_mm_call = pl.pallas_call(
    _mm_scale_body,
    grid=(P // R,),
    in_specs=[
        pl.BlockSpec((R, 128), lambda i: (i, 0)),
        pl.BlockSpec((128, 256), lambda i: (0, 0)),
        pl.BlockSpec((2, R, 128), lambda i: (0, i, 0)),
    ],
    out_specs=pl.BlockSpec((2, R, 128), lambda i: (0, i, 0)),
    out_shape=jax.ShapeDtypeStruct((2, P, 128), jnp.float32),
)

_mid_call = pl.pallas_call(
    _mid_body,
    grid=(P // R,),
    in_specs=[
        pl.BlockSpec((2, R, 128), lambda i: (0, i, 0)),
        pl.BlockSpec((2, R, 128), lambda i: (0, i, 0)),
        pl.BlockSpec((1, 256), lambda i: (0, 0)),
        pl.BlockSpec((256, 256), lambda i: (0, 0)),
    ],
    out_specs=pl.BlockSpec((2, R, 128), lambda i: (0, i, 0)),
    out_shape=jax.ShapeDtypeStruct((2, P, 128), jnp.float32),
)

_out_call = pl.pallas_call(
    _out_body,
    grid=(N // R_OUT,),
    in_specs=[
        pl.BlockSpec((2, R_OUT, 128), lambda i: (0, i, 0)),
        pl.BlockSpec((2, R_OUT, 128), lambda i: (0, i, 0)),
        pl.BlockSpec((1, 256), lambda i: (0, 0)),
    ],
    out_specs=pl.BlockSpec((R_OUT, 256), lambda i: (i, 0)),
    out_shape=jax.ShapeDtypeStruct((N, 256), jnp.float32),
)


def kernel(x, edge_index, W1, b1, W2, b2):
    src = edge_index[0].astype(jnp.int32)
    dst = edge_index[1].astype(jnp.int32)
    pad = EP - E
    src_p = jnp.concatenate([src, jnp.zeros((pad,), jnp.int32)])
    dst_p = jnp.concatenate([dst, jnp.full((pad,), N, jnp.int32)])
    dst_deg = dst_p.reshape(NC, NS, NB_DEG, B)
    dst_agg = dst_p.reshape(NS, NCHUNK, NB_IN, B)
    src_agg = jnp.stack([src_p, src_p + P]).reshape(NC, NS, NCHUNK, NB_IN, B)
    xp = jnp.pad(x, ((0, P - N), (0, 0)))

    deg_kernel, agg_kernel = _sc_kernels()
    ones_hbm = jnp.ones((B, 128), jnp.float32)
    zeros_hbm = jnp.zeros((RPT, 128), jnp.float32)
    degp = deg_kernel(dst_deg, ones_hbm, zeros_hbm)
    t1 = _mm_call(xp, W1, degp)
    agg1 = agg_kernel(t1.reshape(NC * P, 128), src_agg, dst_agg)
    t2 = _mid_call(agg1, degp, b1.reshape(1, 256), W2)
    agg2 = agg_kernel(t2.reshape(NC * P, 128), src_agg, dst_agg)
    return _out_call(agg2, degp, b2.reshape(1, 256))


# speedup vs baseline: 2.7301x; 1.1718x over previous
import functools

import jax
import jax.numpy as jnp
from jax import lax
from jax.experimental import pallas as pl
from jax.experimental.pallas import tpu as pltpu
from jax.experimental.pallas import tpu_sc as plsc

N = 10000
P = 10112
E = 320000
NC, NS = 2, 16
B = 128
EP = 327680
NB_AGG = EP // (NS * B)
NB_DEG = EP // (NC * NS * B)
NCHUNK = 2
NB_IN = NB_AGG // NCHUNK
RPT = P // NS


@functools.cache
def _sc_kernels():
    mesh = plsc.VectorSubcoreMesh(core_axis_name="c", subcore_axis_name="s",
                                  num_cores=NC, num_subcores=NS)

    @functools.partial(
        pl.kernel,
        out_type=jax.ShapeDtypeStruct((NC, P, 128), jnp.float32),
        mesh=mesh,
        scratch_types=[
            pltpu.VMEM((NB_DEG, B), jnp.int32),
            pltpu.VMEM((B, 128), jnp.float32),
            pltpu.VMEM_SHARED((P, 128), jnp.float32),
        ],
    )
    def deg_kernel(dst_hbm, ones_hbm, zeros_hbm, out_hbm, dst_v, ones_v, acc_sh):
        c = lax.axis_index("c")
        s = lax.axis_index("s")
        pltpu.sync_copy(dst_hbm.at[c, s], dst_v)
        pltpu.sync_copy(ones_hbm, ones_v)
        pltpu.sync_copy(zeros_hbm, acc_sh.at[pl.ds(s * RPT, RPT)])
        plsc.subcore_barrier()

        def body(j, _):
            pltpu.sync_copy(ones_v, acc_sh.at[dst_v.at[j]], add=True)
            return 0
        lax.fori_loop(0, NB_DEG, body, 0)

        plsc.subcore_barrier()
        pltpu.sync_copy(acc_sh.at[pl.ds(s * RPT, RPT)],
                        out_hbm.at[c, pl.ds(s * RPT, RPT)])

    @functools.partial(
        pl.kernel,
        out_type=jax.ShapeDtypeStruct((NC, P, 128), jnp.float32),
        mesh=mesh,
        scratch_types=[
            pltpu.VMEM((NB_IN, B), jnp.int32),
            pltpu.VMEM((NB_IN, B), jnp.int32),
            pltpu.VMEM((B, 128), jnp.float32),
            pltpu.VMEM_SHARED((P, 128), jnp.float32),
            pltpu.SemaphoreType.DMA,
        ],
    )
    def agg_kernel(table_hbm, src_hbm, dst_hbm, out_hbm,
                   src_v, dst_v, buf0, acc_sh, semg0):
        c = lax.axis_index("c")
        s = lax.axis_index("s")
        pltpu.sync_copy(table_hbm.at[pl.ds(c * P + s * RPT, RPT)],
                        acc_sh.at[pl.ds(s * RPT, RPT)])
        plsc.subcore_barrier()

        def chunk(ch, _):
            pltpu.sync_copy(src_hbm.at[c, s, ch], src_v)
            pltpu.sync_copy(dst_hbm.at[s, ch], dst_v)

            def body(j, _):
                pltpu.async_copy(table_hbm.at[src_v.at[j]], buf0, semg0).wait()
                pltpu.sync_copy(buf0, acc_sh.at[dst_v.at[j]], add=True)
                return 0
            lax.fori_loop(0, NB_IN, body, 0)
            return 0
        lax.fori_loop(0, NCHUNK, chunk, 0)

        plsc.subcore_barrier()
        pltpu.sync_copy(acc_sh.at[pl.ds(s * RPT, RPT)],
                        out_hbm.at[c, pl.ds(s * RPT, RPT)])

    return deg_kernel, agg_kernel


R = 2528
R_OUT = 2000


def _dinv_of(degp_ref):
    return lax.rsqrt(degp_ref[0, :, 0] + degp_ref[1, :, 0] + 1.0)


def _mm_scale_body(x_ref, w_ref, degp_ref, out_ref):
    dinv = _dinv_of(degp_ref)
    h = jnp.dot(x_ref[...], w_ref[...], preferred_element_type=jnp.float32)
    hp = h * dinv[:, None]
    out_ref[0] = hp[:, :128]
    out_ref[1] = hp[:, 128:]


def _mid_body(agg_ref, degp_ref, b_ref, w_ref, out_ref):
    dinv = _dinv_of(degp_ref)
    h = jnp.concatenate([agg_ref[0], agg_ref[1]], axis=1) * dinv[:, None]
    h = jnp.maximum(h + b_ref[...], 0.0)
    h2 = jnp.dot(h, w_ref[...], preferred_element_type=jnp.float32)
    hp = h2 * dinv[:, None]
    grow = pl.program_id(0) * R + lax.broadcasted_iota(jnp.int32, (R, 1), 0)
    hp = jnp.where(grow < N, hp, 0.0)
    out_ref[0] = hp[:, :128]
    out_ref[1] = hp[:, 128:]


def _out_body(agg_ref, degp_ref, b_ref, out_ref):
    dinv = _dinv_of(degp_ref)
    h = jnp.concatenate([agg_ref[0], agg_ref[1]], axis=1) * dinv[:, None]
    out_ref[...] = jnp.maximum(h + b_ref[...], 0.0)


_mm_call = pl.pallas_call(
    _mm_scale_body,
    grid=(P // R,),
    in_specs=[
        pl.BlockSpec((R, 128), lambda i: (i, 0)),
        pl.BlockSpec((128, 256), lambda i: (0, 0)),
        pl.BlockSpec((2, R, 128), lambda i: (0, i, 0)),
    ],
    out_specs=pl.BlockSpec((2, R, 128), lambda i: (0, i, 0)),
    out_shape=jax.ShapeDtypeStruct((2, P, 128), jnp.float32),
)

_mid_call = pl.pallas_call(
    _mid_body,
    grid=(P // R,),
    in_specs=[
        pl.BlockSpec((2, R, 128), lambda i: (0, i, 0)),
        pl.BlockSpec((2, R, 128), lambda i: (0, i, 0)),
        pl.BlockSpec((1, 256), lambda i: (0, 0)),
        pl.BlockSpec((256, 256), lambda i: (0, 0)),
    ],
    out_specs=pl.BlockSpec((2, R, 128), lambda i: (0, i, 0)),
    out_shape=jax.ShapeDtypeStruct((2, P, 128), jnp.float32),
)

_out_call = pl.pallas_call(
    _out_body,
    grid=(N // R_OUT,),
    in_specs=[
        pl.BlockSpec((2, R_OUT, 128), lambda i: (0, i, 0)),
        pl.BlockSpec((2, R_OUT, 128), lambda i: (0, i, 0)),
        pl.BlockSpec((1, 256), lambda i: (0, 0)),
    ],
    out_specs=pl.BlockSpec((R_OUT, 256), lambda i: (i, 0)),
    out_shape=jax.ShapeDtypeStruct((N, 256), jnp.float32),
)


def kernel(x, edge_index, W1, b1, W2, b2):
    src = edge_index[0].astype(jnp.int32)
    dst = edge_index[1].astype(jnp.int32)
    pad = EP - E
    pad_idx = jnp.arange(pad, dtype=jnp.int32)
    src_p = jnp.concatenate([src, jnp.full((pad,), N, jnp.int32)])
    dst_pa = jnp.concatenate([dst, pad_idx % N])
    dst_pd = jnp.concatenate([dst, N + pad_idx % (P - N)])
    dst_deg = dst_pd.reshape(NC, NS, NB_DEG, B)
    dst_agg = dst_pa.reshape(NS, NCHUNK, NB_IN, B)
    src_agg = jnp.stack([src_p, src_p + P]).reshape(NC, NS, NCHUNK, NB_IN, B)
    xp = jnp.pad(x, ((0, P - N), (0, 0)))

    deg_kernel, agg_kernel = _sc_kernels()
    ones_hbm = jnp.ones((B, 128), jnp.float32)
    zeros_hbm = jnp.zeros((RPT, 128), jnp.float32)
    degp = deg_kernel(dst_deg, ones_hbm, zeros_hbm)
    t1 = _mm_call(xp, W1, degp)
    agg1 = agg_kernel(t1.reshape(NC * P, 128), src_agg, dst_agg)
    t2 = _mid_call(agg1, degp, b1.reshape(1, 256), W2)
    agg2 = agg_kernel(t2.reshape(NC * P, 128), src_agg, dst_agg)
    return _out_call(agg2, degp, b2.reshape(1, 256))


# speedup vs baseline: 4.5825x; 1.6785x over previous
import functools

import jax
import jax.numpy as jnp
from jax import lax
from jax.experimental import pallas as pl
from jax.experimental.pallas import tpu as pltpu
from jax.experimental.pallas import tpu_sc as plsc

N = 10000
P = 10112
E = 320000
NC, NS = 2, 16
B = 128
EP = 327680
NB_AGG = EP // (NS * B)
NB_DEG = EP // (NC * NS * B)
NCHUNK = 2
NB_IN = NB_AGG // NCHUNK
RPT = P // NS


@functools.cache
def _sc_kernels():
    mesh = plsc.VectorSubcoreMesh(core_axis_name="c", subcore_axis_name="s",
                                  num_cores=NC, num_subcores=NS)

    @functools.partial(
        pl.kernel,
        out_type=jax.ShapeDtypeStruct((NC, P, 128), jnp.float32),
        mesh=mesh,
        scratch_types=[
            pltpu.VMEM((NB_DEG, B), jnp.int32),
            pltpu.VMEM((B, 128), jnp.float32),
            pltpu.VMEM_SHARED((P, 128), jnp.float32),
        ],
    )
    def deg_kernel(dst_hbm, ones_hbm, zeros_hbm, out_hbm, dst_v, ones_v, acc_sh):
        c = lax.axis_index("c")
        s = lax.axis_index("s")
        pltpu.sync_copy(dst_hbm.at[c, s], dst_v)
        pltpu.sync_copy(ones_hbm, ones_v)
        pltpu.sync_copy(zeros_hbm, acc_sh.at[pl.ds(s * RPT, RPT)])
        plsc.subcore_barrier()

        def body(j, _):
            pltpu.sync_copy(ones_v, acc_sh.at[dst_v.at[j]], add=True)
            return 0
        lax.fori_loop(0, NB_DEG, body, 0)

        plsc.subcore_barrier()
        pltpu.sync_copy(acc_sh.at[pl.ds(s * RPT, RPT)],
                        out_hbm.at[c, pl.ds(s * RPT, RPT)])

    @functools.partial(
        pl.kernel,
        out_type=jax.ShapeDtypeStruct((NC, P, 128), jnp.float32),
        mesh=mesh,
        scratch_types=[
            pltpu.VMEM((NB_IN, B), jnp.int32),
            pltpu.VMEM((NB_IN, B), jnp.int32),
            pltpu.VMEM((B, 128), jnp.float32),
            pltpu.VMEM_SHARED((P, 128), jnp.float32),
            pltpu.SemaphoreType.DMA,
        ],
    )
    def agg_kernel(table_hbm, src_hbm, dst_hbm, out_hbm,
                   src_v, dst_v, buf0, acc_sh, semg0):
        c = lax.axis_index("c")
        s = lax.axis_index("s")
        pltpu.sync_copy(table_hbm.at[pl.ds(c * P + s * RPT, RPT)],
                        acc_sh.at[pl.ds(s * RPT, RPT)])
        plsc.subcore_barrier()

        def chunk(ch, _):
            pltpu.sync_copy(src_hbm.at[c, s, ch], src_v)
            pltpu.sync_copy(dst_hbm.at[s, ch], dst_v)

            def body(j, _):
                pltpu.async_copy(table_hbm.at[src_v.at[j]], buf0, semg0).wait()
                pltpu.sync_copy(buf0, acc_sh.at[dst_v.at[j]], add=True)
                return 0
            lax.fori_loop(0, NB_IN, body, 0)
            return 0
        lax.fori_loop(0, NCHUNK, chunk, 0)

        plsc.subcore_barrier()
        pltpu.sync_copy(acc_sh.at[pl.ds(s * RPT, RPT)],
                        out_hbm.at[c, pl.ds(s * RPT, RPT)])

    return deg_kernel, agg_kernel


R = 2528
R_OUT = 2000


def _dinv_of(degp_ref):
    return lax.rsqrt(degp_ref[0, :, 0] + degp_ref[1, :, 0] + 1.0)


def _mm_scale_body(x_ref, w_ref, degp_ref, out_ref):
    dinv = _dinv_of(degp_ref)
    h = jnp.dot(x_ref[...], w_ref[...], preferred_element_type=jnp.float32)
    hp = h * dinv[:, None]
    out_ref[0] = hp[:, :128]
    out_ref[1] = hp[:, 128:]


def _mid_body(agg_ref, degp_ref, b_ref, w_ref, out_ref):
    dinv = _dinv_of(degp_ref)
    h = jnp.concatenate([agg_ref[0], agg_ref[1]], axis=1) * dinv[:, None]
    h = jnp.maximum(h + b_ref[...], 0.0)
    h2 = jnp.dot(h, w_ref[...], preferred_element_type=jnp.float32)
    hp = h2 * dinv[:, None]
    grow = pl.program_id(0) * R + lax.broadcasted_iota(jnp.int32, (R, 1), 0)
    hp = jnp.where(grow < N, hp, 0.0)
    out_ref[0] = hp[:, :128]
    out_ref[1] = hp[:, 128:]


def _out_body(agg_ref, degp_ref, b_ref, out_ref):
    dinv = _dinv_of(degp_ref)
    h = jnp.concatenate([agg_ref[0], agg_ref[1]], axis=1) * dinv[:, None]
    out_ref[...] = jnp.maximum(h + b_ref[...], 0.0)


_mm_call = pl.pallas_call(
    _mm_scale_body,
    grid=(P // R,),
    in_specs=[
        pl.BlockSpec((R, 128), lambda i: (i, 0)),
        pl.BlockSpec((128, 256), lambda i: (0, 0)),
        pl.BlockSpec((2, R, 128), lambda i: (0, i, 0)),
    ],
    out_specs=pl.BlockSpec((2, R, 128), lambda i: (0, i, 0)),
    out_shape=jax.ShapeDtypeStruct((2, P, 128), jnp.float32),
)

_mid_call = pl.pallas_call(
    _mid_body,
    grid=(P // R,),
    in_specs=[
        pl.BlockSpec((2, R, 128), lambda i: (0, i, 0)),
        pl.BlockSpec((2, R, 128), lambda i: (0, i, 0)),
        pl.BlockSpec((1, 256), lambda i: (0, 0)),
        pl.BlockSpec((256, 256), lambda i: (0, 0)),
    ],
    out_specs=pl.BlockSpec((2, R, 128), lambda i: (0, i, 0)),
    out_shape=jax.ShapeDtypeStruct((2, P, 128), jnp.float32),
)

_out_call = pl.pallas_call(
    _out_body,
    grid=(N // R_OUT,),
    in_specs=[
        pl.BlockSpec((2, R_OUT, 128), lambda i: (0, i, 0)),
        pl.BlockSpec((2, R_OUT, 128), lambda i: (0, i, 0)),
        pl.BlockSpec((1, 256), lambda i: (0, 0)),
    ],
    out_specs=pl.BlockSpec((R_OUT, 256), lambda i: (i, 0)),
    out_shape=jax.ShapeDtypeStruct((N, 256), jnp.float32),
)


def kernel(x, edge_index, W1, b1, W2, b2):
    src = edge_index[0].astype(jnp.int32)
    dst = edge_index[1].astype(jnp.int32)
    pad = EP - E
    pad_idx = jnp.arange(pad, dtype=jnp.int32)
    src_p = jnp.concatenate([src, N + pad_idx % (P - N)])
    dst_pa = jnp.concatenate([dst, pad_idx % N])
    dst_pd = jnp.concatenate([dst, N + pad_idx % (P - N)])
    dst_deg = dst_pd.reshape(NC, NS, NB_DEG, B)
    dst_agg = dst_pa.reshape(NS, NCHUNK, NB_IN, B)
    src_agg = jnp.stack([src_p, src_p + P]).reshape(NC, NS, NCHUNK, NB_IN, B)
    xp = jnp.pad(x, ((0, P - N), (0, 0)))

    deg_kernel, agg_kernel = _sc_kernels()
    ones_hbm = jnp.ones((B, 128), jnp.float32)
    zeros_hbm = jnp.zeros((RPT, 128), jnp.float32)
    degp = deg_kernel(dst_deg, ones_hbm, zeros_hbm)
    t1 = _mm_call(xp, W1, degp)
    agg1 = agg_kernel(t1.reshape(NC * P, 128), src_agg, dst_agg)
    t2 = _mid_call(agg1, degp, b1.reshape(1, 256), W2)
    agg2 = agg_kernel(t2.reshape(NC * P, 128), src_agg, dst_agg)
    return _out_call(agg2, degp, b2.reshape(1, 256))


# speedup vs baseline: 6.7660x; 1.4765x over previous
import functools

import jax
import jax.numpy as jnp
from jax import lax
from jax.experimental import pallas as pl
from jax.experimental.pallas import tpu as pltpu
from jax.experimental.pallas import tpu_sc as plsc

N = 10000
P = 10112
E = 320000
NC, NS = 2, 16
B = 128
EP = 327680
NB_AGG = EP // (NS * B)
NB_DEG = EP // (NC * NS * B)
NCHUNK = 4
NB_IN = NB_AGG // NCHUNK
NB2 = NB_IN // 2
RPT = P // NS


@functools.cache
def _sc_kernels():
    mesh = plsc.VectorSubcoreMesh(core_axis_name="c", subcore_axis_name="s",
                                  num_cores=NC, num_subcores=NS)

    @functools.partial(
        pl.kernel,
        out_type=jax.ShapeDtypeStruct((NC, P, 128), jnp.float32),
        mesh=mesh,
        scratch_types=[
            pltpu.VMEM((NB_DEG, B), jnp.int32),
            pltpu.VMEM((B, 128), jnp.float32),
            pltpu.VMEM_SHARED((P, 128), jnp.float32),
        ],
    )
    def deg_kernel(dst_hbm, ones_hbm, zeros_hbm, out_hbm, dst_v, ones_v, acc_sh):
        c = lax.axis_index("c")
        s = lax.axis_index("s")
        pltpu.sync_copy(dst_hbm.at[c, s], dst_v)
        pltpu.sync_copy(ones_hbm, ones_v)
        pltpu.sync_copy(zeros_hbm, acc_sh.at[pl.ds(s * RPT, RPT)])
        plsc.subcore_barrier()

        def body(j, _):
            pltpu.sync_copy(ones_v, acc_sh.at[dst_v.at[j]], add=True)
            return 0
        lax.fori_loop(0, NB_DEG, body, 0)

        plsc.subcore_barrier()
        pltpu.sync_copy(acc_sh.at[pl.ds(s * RPT, RPT)],
                        out_hbm.at[c, pl.ds(s * RPT, RPT)])

    @functools.partial(
        pl.kernel,
        out_type=jax.ShapeDtypeStruct((NC, P, 128), jnp.float32),
        mesh=mesh,
        scratch_types=[
            pltpu.VMEM((NB_IN, B), jnp.int32),
            pltpu.VMEM((NB_IN, B), jnp.int32),
            pltpu.VMEM((B, 128), jnp.float32),
            pltpu.VMEM((B, 128), jnp.float32),
            pltpu.VMEM_SHARED((P, 128), jnp.float32),
            pltpu.SemaphoreType.DMA,
            pltpu.SemaphoreType.DMA,
        ],
    )
    def agg_kernel(table_hbm, src_hbm, dst_hbm, out_hbm,
                   src_v, dst_v, buf0, buf1, acc_sh, semg0, semg1):
        c = lax.axis_index("c")
        s = lax.axis_index("s")
        pltpu.sync_copy(table_hbm.at[pl.ds(c * P + s * RPT, RPT)],
                        acc_sh.at[pl.ds(s * RPT, RPT)])
        plsc.subcore_barrier()

        def chunk(ch, _):
            pltpu.sync_copy(src_hbm.at[c, s, ch], src_v)
            pltpu.sync_copy(dst_hbm.at[s, ch], dst_v)
            pltpu.async_copy(table_hbm.at[src_v.at[0]], buf0, semg0)

            def body(k, _):
                j0 = 2 * k
                j1 = j0 + 1
                pltpu.async_copy(table_hbm.at[src_v.at[j1]], buf1, semg1)
                pltpu.make_async_copy(table_hbm.at[src_v.at[j0]],
                                      buf0, semg0).wait()
                pltpu.sync_copy(buf0, acc_sh.at[dst_v.at[j0]], add=True)

                @pl.when(k < NB2 - 1)
                def _():
                    pltpu.async_copy(table_hbm.at[src_v.at[j0 + 2]],
                                     buf0, semg0)
                pltpu.make_async_copy(table_hbm.at[src_v.at[j1]],
                                      buf1, semg1).wait()
                pltpu.sync_copy(buf1, acc_sh.at[dst_v.at[j1]], add=True)
                return 0
            lax.fori_loop(0, NB2, body, 0)
            return 0
        lax.fori_loop(0, NCHUNK, chunk, 0)

        plsc.subcore_barrier()
        pltpu.sync_copy(acc_sh.at[pl.ds(s * RPT, RPT)],
                        out_hbm.at[c, pl.ds(s * RPT, RPT)])

    return deg_kernel, agg_kernel


R = 2528
R_OUT = 2000


def _dinv_of(degp_ref):
    return lax.rsqrt(degp_ref[0, :, 0] + degp_ref[1, :, 0] + 1.0)


def _mm_scale_body(x_ref, w_ref, degp_ref, out_ref):
    dinv = _dinv_of(degp_ref)
    h = jnp.dot(x_ref[...], w_ref[...], preferred_element_type=jnp.float32)
    hp = h * dinv[:, None]
    out_ref[0] = hp[:, :128]
    out_ref[1] = hp[:, 128:]


def _mid_body(agg_ref, degp_ref, b_ref, w_ref, out_ref):
    dinv = _dinv_of(degp_ref)
    h = jnp.concatenate([agg_ref[0], agg_ref[1]], axis=1) * dinv[:, None]
    h = jnp.maximum(h + b_ref[...], 0.0)
    h2 = jnp.dot(h, w_ref[...], preferred_element_type=jnp.float32)
    hp = h2 * dinv[:, None]
    grow = pl.program_id(0) * R + lax.broadcasted_iota(jnp.int32, (R, 1), 0)
    hp = jnp.where(grow < N, hp, 0.0)
    out_ref[0] = hp[:, :128]
    out_ref[1] = hp[:, 128:]


def _out_body(agg_ref, degp_ref, b_ref, out_ref):
    dinv = _dinv_of(degp_ref)
    h = jnp.concatenate([agg_ref[0], agg_ref[1]], axis=1) * dinv[:, None]
    out_ref[...] = jnp.maximum(h + b_ref[...], 0.0)


_mm_call = pl.pallas_call(
    _mm_scale_body,
    grid=(P // R,),
    in_specs=[
        pl.BlockSpec((R, 128), lambda i: (i, 0)),
        pl.BlockSpec((128, 256), lambda i: (0, 0)),
        pl.BlockSpec((2, R, 128), lambda i: (0, i, 0)),
    ],
    out_specs=pl.BlockSpec((2, R, 128), lambda i: (0, i, 0)),
    out_shape=jax.ShapeDtypeStruct((2, P, 128), jnp.float32),
)

_mid_call = pl.pallas_call(
    _mid_body,
    grid=(P // R,),
    in_specs=[
        pl.BlockSpec((2, R, 128), lambda i: (0, i, 0)),
        pl.BlockSpec((2, R, 128), lambda i: (0, i, 0)),
        pl.BlockSpec((1, 256), lambda i: (0, 0)),
        pl.BlockSpec((256, 256), lambda i: (0, 0)),
    ],
    out_specs=pl.BlockSpec((2, R, 128), lambda i: (0, i, 0)),
    out_shape=jax.ShapeDtypeStruct((2, P, 128), jnp.float32),
)

_out_call = pl.pallas_call(
    _out_body,
    grid=(N // R_OUT,),
    in_specs=[
        pl.BlockSpec((2, R_OUT, 128), lambda i: (0, i, 0)),
        pl.BlockSpec((2, R_OUT, 128), lambda i: (0, i, 0)),
        pl.BlockSpec((1, 256), lambda i: (0, 0)),
    ],
    out_specs=pl.BlockSpec((R_OUT, 256), lambda i: (i, 0)),
    out_shape=jax.ShapeDtypeStruct((N, 256), jnp.float32),
)


def kernel(x, edge_index, W1, b1, W2, b2):
    src = edge_index[0].astype(jnp.int32)
    dst = edge_index[1].astype(jnp.int32)
    pad = EP - E
    pad_idx = jnp.arange(pad, dtype=jnp.int32)
    src_p = jnp.concatenate([src, N + pad_idx % (P - N)])
    dst_pa = jnp.concatenate([dst, pad_idx % N])
    dst_pd = jnp.concatenate([dst, N + pad_idx % (P - N)])
    dst_deg = dst_pd.reshape(NC, NS, NB_DEG, B)
    dst_agg = dst_pa.reshape(NS, NCHUNK, NB_IN, B)
    src_agg = jnp.stack([src_p, src_p + P]).reshape(NC, NS, NCHUNK, NB_IN, B)
    xp = jnp.pad(x, ((0, P - N), (0, 0)))

    deg_kernel, agg_kernel = _sc_kernels()
    ones_hbm = jnp.ones((B, 128), jnp.float32)
    zeros_hbm = jnp.zeros((RPT, 128), jnp.float32)
    degp = deg_kernel(dst_deg, ones_hbm, zeros_hbm)
    t1 = _mm_call(xp, W1, degp)
    agg1 = agg_kernel(t1.reshape(NC * P, 128), src_agg, dst_agg)
    t2 = _mid_call(agg1, degp, b1.reshape(1, 256), W2)
    agg2 = agg_kernel(t2.reshape(NC * P, 128), src_agg, dst_agg)
    return _out_call(agg2, degp, b2.reshape(1, 256))
